# Initial kernel scaffold; baseline (speedup 1.0000x reference)
import jax
import jax.numpy as jnp
from jax.experimental import pallas as pl


def kernel(x, edge_index, W1, b1, W2, b2, W3, b3, Cw1, Cb1, Cw2, Cb2, Cw3, Cb3):
    raise NotImplementedError("write your pallas kernel here")


# speedup vs baseline: 7.0685x; 7.0685x over previous
import functools

import jax
import jax.numpy as jnp
from jax import lax
from jax.experimental import pallas as pl
from jax.experimental.pallas import tpu as pltpu
from jax.experimental.pallas import tpu_sc as plsc

N = 10000
E = 320000
F_IN = 128
H = 256
NC = 2
NS = 16
HH = H // NC
C = 80
EPT = E // NS
EC = EPT // C
EPW = E // (NC * NS)
EDC = EPW // C
NP = 10240
RPT = NP // NS
RW = 128

_MESH = plsc.VectorSubcoreMesh(
    core_axis_name="c", subcore_axis_name="s", num_cores=NC, num_subcores=NS
)


@functools.partial(
    pl.kernel,
    out_type=jax.ShapeDtypeStruct((NC * NP, HH), jnp.float32),
    mesh=_MESH,
    scratch_types=[
        pltpu.VMEM((2 * C,), jnp.int32),
        pltpu.VMEM((C, HH), jnp.float32),
        pltpu.VMEM((RW, HH), jnp.float32),
        pltpu.VMEM_SHARED((NP, HH), jnp.float32),
        pltpu.SemaphoreType.DMA,
    ],
)
def _sc_prop(hp2_hbm, idx_hbm, zeros_hbm, out_hbm, idx_v, rows_v, buf_v,
             acc_sh, sem):
    c = lax.axis_index("c")
    s = lax.axis_index("s")
    pltpu.sync_copy(zeros_hbm, buf_v)
    for k in range(RPT // RW):
        pltpu.sync_copy(buf_v, acc_sh.at[pl.ds(s * RPT + k * RW, RW)])
    plsc.subcore_barrier()
    base = (c * NS + s) * EC

    def body(g, carry):
        pltpu.sync_copy(idx_hbm.at[pl.ds((base + g) * 2 * C, 2 * C)], idx_v)
        pltpu.async_copy(
            hp2_hbm.at[idx_v.at[pl.ds(0, C)]], rows_v, sem
        ).wait()
        pltpu.sync_copy(
            rows_v, acc_sh.at[idx_v.at[pl.ds(C, C)]], add=True
        )
        return carry

    lax.fori_loop(0, EC, body, 0)
    plsc.subcore_barrier()
    for k in range(RPT // RW):
        r = s * RPT + k * RW
        pltpu.sync_copy(acc_sh.at[pl.ds(r, RW)], buf_v)
        pltpu.sync_copy(buf_v, out_hbm.at[pl.ds(c * NP + r, RW)])


_BM = 2000
_GRID = N // _BM


def _tc_first_body(deg_ref, x_ref, w_ref, dinv_ref, hp_ref):
    deg = deg_ref[:, 0:1] + 1.0
    dinv = lax.rsqrt(deg)
    dinv_ref[...] = dinv
    xw = jnp.dot(x_ref[...], w_ref[...],
                 preferred_element_type=jnp.float32,
                 precision=lax.Precision.HIGHEST)
    hp_ref[...] = dinv * xw


def _tc_first(deg_wide, x, W1):
    return pl.pallas_call(
        _tc_first_body,
        grid=(_GRID,),
        in_specs=[
            pl.BlockSpec((_BM, HH), lambda i: (i, 0)),
            pl.BlockSpec((_BM, F_IN), lambda i: (i, 0)),
            pl.BlockSpec((F_IN, H), lambda i: (0, 0)),
        ],
        out_specs=[
            pl.BlockSpec((_BM, 1), lambda i: (i, 0)),
            pl.BlockSpec((_BM, H), lambda i: (i, 0)),
        ],
        out_shape=[
            jax.ShapeDtypeStruct((N, 1), jnp.float32),
            jax.ShapeDtypeStruct((N, H), jnp.float32),
        ],
    )(deg_wide, x, W1)


def _tc_mid_body(acc_ref, hp_ref, dinv_ref, b_ref, w_ref, out_ref):
    accfull = jnp.concatenate([acc_ref[0], acc_ref[1]], axis=1)
    dinv = dinv_ref[...]
    h = jnp.maximum(dinv * (accfull + hp_ref[...]) + b_ref[...], 0.0)
    hw = jnp.dot(h, w_ref[...], preferred_element_type=jnp.float32,
                 precision=lax.Precision.HIGHEST)
    out_ref[...] = dinv * hw


def _tc_mid(acc, hp, dinv, b_prev, W_next):
    return pl.pallas_call(
        _tc_mid_body,
        grid=(_GRID,),
        in_specs=[
            pl.BlockSpec((NC, _BM, HH), lambda i: (0, i, 0)),
            pl.BlockSpec((_BM, H), lambda i: (i, 0)),
            pl.BlockSpec((_BM, 1), lambda i: (i, 0)),
            pl.BlockSpec((1, H), lambda i: (0, 0)),
            pl.BlockSpec((H, H), lambda i: (0, 0)),
        ],
        out_specs=pl.BlockSpec((_BM, H), lambda i: (i, 0)),
        out_shape=jax.ShapeDtypeStruct((N, H), jnp.float32),
    )(acc, hp, dinv, b_prev, W_next)


def _tc_final_body(acc_ref, hp_ref, dinv_ref, b_ref, cw1_ref, cb1_ref,
                   cw2_ref, cb2_ref, cw3_ref, cb3_ref, out_ref,
                   sum_s, max_s):
    i = pl.program_id(0)
    accfull = jnp.concatenate([acc_ref[0], acc_ref[1]], axis=1)
    h = jnp.maximum(dinv_ref[...] * (accfull + hp_ref[...]) + b_ref[...], 0.0)
    psum = jnp.sum(h, axis=0, keepdims=True)
    pmax = jnp.max(h, axis=0, keepdims=True)

    @pl.when(i == 0)
    def _():
        sum_s[...] = psum
        max_s[...] = pmax

    @pl.when(i > 0)
    def _():
        sum_s[...] += psum
        max_s[...] = jnp.maximum(max_s[...], pmax)

    @pl.when(i == _GRID - 1)
    def _():
        g = jnp.concatenate([sum_s[...] * (1.0 / N), max_s[...]], axis=1)
        z = jnp.maximum(
            jnp.dot(g, cw1_ref[...], preferred_element_type=jnp.float32,
                    precision=lax.Precision.HIGHEST) + cb1_ref[...], 0.0)
        z = jnp.maximum(
            jnp.dot(z, cw2_ref[...], preferred_element_type=jnp.float32,
                    precision=lax.Precision.HIGHEST) + cb2_ref[...], 0.0)
        out_ref[...] = jnp.dot(
            z, cw3_ref[...], preferred_element_type=jnp.float32,
            precision=lax.Precision.HIGHEST) + cb3_ref[...]


def _tc_final(acc, hp, dinv, b3, Cw1, Cb1, Cw2, Cb2, Cw3, Cb3):
    return pl.pallas_call(
        _tc_final_body,
        grid=(_GRID,),
        in_specs=[
            pl.BlockSpec((NC, _BM, HH), lambda i: (0, i, 0)),
            pl.BlockSpec((_BM, H), lambda i: (i, 0)),
            pl.BlockSpec((_BM, 1), lambda i: (i, 0)),
            pl.BlockSpec((1, H), lambda i: (0, 0)),
            pl.BlockSpec((2 * H, 512), lambda i: (0, 0)),
            pl.BlockSpec((1, 512), lambda i: (0, 0)),
            pl.BlockSpec((512, 256), lambda i: (0, 0)),
            pl.BlockSpec((1, 256), lambda i: (0, 0)),
            pl.BlockSpec((256, 5), lambda i: (0, 0)),
            pl.BlockSpec((1, 5), lambda i: (0, 0)),
        ],
        out_specs=pl.BlockSpec((1, 5), lambda i: (0, 0)),
        out_shape=jax.ShapeDtypeStruct((1, 5), jnp.float32),
        scratch_shapes=[
            pltpu.VMEM((1, H), jnp.float32),
            pltpu.VMEM((1, H), jnp.float32),
        ],
    )(acc, hp, dinv, b3, Cw1, Cb1, Cw2, Cb2, Cw3, Cb3)


def kernel(x, edge_index, W1, b1, W2, b2, W3, b3,
           Cw1, Cb1, Cw2, Cb2, Cw3, Cb3):
    src = edge_index[0]
    dst = edge_index[1]
    srcs = jnp.stack([2 * src, 2 * src + 1], axis=0).reshape(NC, NS, EC, C)
    dsts = jnp.broadcast_to(dst.reshape(1, NS, EC, C), (NC, NS, EC, C))
    idx_all = jnp.concatenate([srcs, dsts], axis=-1).reshape(-1)
    zerosHH = jnp.zeros((RW, HH), jnp.float32)

    ones2N = jnp.ones((NC * N, HH), jnp.float32)
    deg_wide = _sc_prop(ones2N, idx_all, zerosHH).reshape(NC, NP, HH)[0, :N, :]
    dinv, hp = _tc_first(deg_wide, x, W1)

    hp2 = hp.reshape(NC * N, HH)
    acc = _sc_prop(hp2, idx_all, zerosHH).reshape(NC, NP, HH)[:, :N, :]
    hp = _tc_mid(acc, hp, dinv, b1.reshape(1, H), W2)

    hp2 = hp.reshape(NC * N, HH)
    acc = _sc_prop(hp2, idx_all, zerosHH).reshape(NC, NP, HH)[:, :N, :]
    hp = _tc_mid(acc, hp, dinv, b2.reshape(1, H), W3)

    hp2 = hp.reshape(NC * N, HH)
    acc = _sc_prop(hp2, idx_all, zerosHH).reshape(NC, NP, HH)[:, :N, :]
    return _tc_final(acc, hp, dinv, b3.reshape(1, H),
                     Cw1, Cb1.reshape(1, 512), Cw2, Cb2.reshape(1, 256),
                     Cw3, Cb3.reshape(1, 5))


# speedup vs baseline: 13.9272x; 1.9703x over previous
import functools

import jax
import jax.numpy as jnp
from jax import lax
from jax.experimental import pallas as pl
from jax.experimental.pallas import tpu as pltpu
from jax.experimental.pallas import tpu_sc as plsc

N = 10000
E = 320000
F_IN = 128
H = 256
NC = 2
NS = 16
HH = H // NC
C = 80
EPT = E // NS
EC = EPT // C
EPW = E // (NC * NS)
EDC = EPW // C
NP = 10240
RPT = NP // NS
RW = 128

_MESH = plsc.VectorSubcoreMesh(
    core_axis_name="c", subcore_axis_name="s", num_cores=NC, num_subcores=NS
)


SB = 25
NSB = EC // SB


@functools.partial(
    pl.kernel,
    out_type=jax.ShapeDtypeStruct((NC * NP, HH), jnp.float32),
    mesh=_MESH,
    scratch_types=[
        pltpu.VMEM((SB * 2 * C,), jnp.int32),
        pltpu.VMEM((C, HH), jnp.float32),
        pltpu.VMEM((C, HH), jnp.float32),
        pltpu.VMEM_SHARED((NP, HH), jnp.float32),
        pltpu.SemaphoreType.DMA,
        pltpu.SemaphoreType.DMA,
    ],
)
def _sc_prop(hp2_hbm, idx_hbm, zeros_hbm, out_hbm, idxb, rows0, rows1,
             acc_sh, sem0, sem1):
    c = lax.axis_index("c")
    s = lax.axis_index("s")
    rows = (rows0, rows1)
    sems = (sem0, sem1)
    pltpu.sync_copy(zeros_hbm, rows0)
    for k in range(RPT // C):
        pltpu.sync_copy(rows0, acc_sh.at[pl.ds(s * RPT + k * C, C)])
    plsc.subcore_barrier()
    base = (c * NS + s) * EC

    def outer(o, carry):
        pltpu.sync_copy(
            idx_hbm.at[pl.ds((base + o * SB) * 2 * C, SB * 2 * C)], idxb
        )
        desc = pltpu.async_copy(
            hp2_hbm.at[idxb.at[pl.ds(0, C)]], rows0, sem0
        )
        for j in range(SB):
            if j + 1 < SB:
                nxt = pltpu.async_copy(
                    hp2_hbm.at[idxb.at[pl.ds((j + 1) * 2 * C, C)]],
                    rows[(j + 1) % 2], sems[(j + 1) % 2],
                )
            desc.wait()
            pltpu.sync_copy(
                rows[j % 2], acc_sh.at[idxb.at[pl.ds(j * 2 * C + C, C)]],
                add=True,
            )
            if j + 1 < SB:
                desc = nxt
        return carry

    lax.fori_loop(0, NSB, outer, 0)
    plsc.subcore_barrier()
    for k in range(RPT // C):
        r = s * RPT + k * C
        pltpu.sync_copy(acc_sh.at[pl.ds(r, C)], rows0)
        pltpu.sync_copy(rows0, out_hbm.at[pl.ds(c * NP + r, C)])


_BM = 2000
_GRID = N // _BM


def _tc_first_body(deg_ref, x_ref, w_ref, dinv_ref, hp_ref):
    deg = deg_ref[:, 0:1] + 1.0
    dinv = lax.rsqrt(deg)
    dinv_ref[...] = dinv
    xw = jnp.dot(x_ref[...], w_ref[...],
                 preferred_element_type=jnp.float32,
                 precision=lax.Precision.HIGHEST)
    hp_ref[...] = dinv * xw


def _tc_first(deg_wide, x, W1):
    return pl.pallas_call(
        _tc_first_body,
        grid=(_GRID,),
        in_specs=[
            pl.BlockSpec((_BM, HH), lambda i: (i, 0)),
            pl.BlockSpec((_BM, F_IN), lambda i: (i, 0)),
            pl.BlockSpec((F_IN, H), lambda i: (0, 0)),
        ],
        out_specs=[
            pl.BlockSpec((_BM, 1), lambda i: (i, 0)),
            pl.BlockSpec((_BM, H), lambda i: (i, 0)),
        ],
        out_shape=[
            jax.ShapeDtypeStruct((N, 1), jnp.float32),
            jax.ShapeDtypeStruct((N, H), jnp.float32),
        ],
    )(deg_wide, x, W1)


def _tc_mid_body(acc_ref, hp_ref, dinv_ref, b_ref, w_ref, out_ref):
    accfull = jnp.concatenate([acc_ref[0], acc_ref[1]], axis=1)
    dinv = dinv_ref[...]
    h = jnp.maximum(dinv * (accfull + hp_ref[...]) + b_ref[...], 0.0)
    hw = jnp.dot(h, w_ref[...], preferred_element_type=jnp.float32,
                 precision=lax.Precision.HIGHEST)
    out_ref[...] = dinv * hw


def _tc_mid(acc, hp, dinv, b_prev, W_next):
    return pl.pallas_call(
        _tc_mid_body,
        grid=(_GRID,),
        in_specs=[
            pl.BlockSpec((NC, _BM, HH), lambda i: (0, i, 0)),
            pl.BlockSpec((_BM, H), lambda i: (i, 0)),
            pl.BlockSpec((_BM, 1), lambda i: (i, 0)),
            pl.BlockSpec((1, H), lambda i: (0, 0)),
            pl.BlockSpec((H, H), lambda i: (0, 0)),
        ],
        out_specs=pl.BlockSpec((_BM, H), lambda i: (i, 0)),
        out_shape=jax.ShapeDtypeStruct((N, H), jnp.float32),
    )(acc, hp, dinv, b_prev, W_next)


def _tc_final_body(acc_ref, hp_ref, dinv_ref, b_ref, cw1_ref, cb1_ref,
                   cw2_ref, cb2_ref, cw3_ref, cb3_ref, out_ref,
                   sum_s, max_s):
    i = pl.program_id(0)
    accfull = jnp.concatenate([acc_ref[0], acc_ref[1]], axis=1)
    h = jnp.maximum(dinv_ref[...] * (accfull + hp_ref[...]) + b_ref[...], 0.0)
    psum = jnp.sum(h, axis=0, keepdims=True)
    pmax = jnp.max(h, axis=0, keepdims=True)

    @pl.when(i == 0)
    def _():
        sum_s[...] = psum
        max_s[...] = pmax

    @pl.when(i > 0)
    def _():
        sum_s[...] += psum
        max_s[...] = jnp.maximum(max_s[...], pmax)

    @pl.when(i == _GRID - 1)
    def _():
        g = jnp.concatenate([sum_s[...] * (1.0 / N), max_s[...]], axis=1)
        z = jnp.maximum(
            jnp.dot(g, cw1_ref[...], preferred_element_type=jnp.float32,
                    precision=lax.Precision.HIGHEST) + cb1_ref[...], 0.0)
        z = jnp.maximum(
            jnp.dot(z, cw2_ref[...], preferred_element_type=jnp.float32,
                    precision=lax.Precision.HIGHEST) + cb2_ref[...], 0.0)
        out_ref[...] = jnp.dot(
            z, cw3_ref[...], preferred_element_type=jnp.float32,
            precision=lax.Precision.HIGHEST) + cb3_ref[...]


def _tc_final(acc, hp, dinv, b3, Cw1, Cb1, Cw2, Cb2, Cw3, Cb3):
    return pl.pallas_call(
        _tc_final_body,
        grid=(_GRID,),
        in_specs=[
            pl.BlockSpec((NC, _BM, HH), lambda i: (0, i, 0)),
            pl.BlockSpec((_BM, H), lambda i: (i, 0)),
            pl.BlockSpec((_BM, 1), lambda i: (i, 0)),
            pl.BlockSpec((1, H), lambda i: (0, 0)),
            pl.BlockSpec((2 * H, 512), lambda i: (0, 0)),
            pl.BlockSpec((1, 512), lambda i: (0, 0)),
            pl.BlockSpec((512, 256), lambda i: (0, 0)),
            pl.BlockSpec((1, 256), lambda i: (0, 0)),
            pl.BlockSpec((256, 5), lambda i: (0, 0)),
            pl.BlockSpec((1, 5), lambda i: (0, 0)),
        ],
        out_specs=pl.BlockSpec((1, 5), lambda i: (0, 0)),
        out_shape=jax.ShapeDtypeStruct((1, 5), jnp.float32),
        scratch_shapes=[
            pltpu.VMEM((1, H), jnp.float32),
            pltpu.VMEM((1, H), jnp.float32),
        ],
    )(acc, hp, dinv, b3, Cw1, Cb1, Cw2, Cb2, Cw3, Cb3)


def kernel(x, edge_index, W1, b1, W2, b2, W3, b3,
           Cw1, Cb1, Cw2, Cb2, Cw3, Cb3):
    src = edge_index[0]
    dst = edge_index[1]
    srcs = jnp.stack([2 * src, 2 * src + 1], axis=0).reshape(NC, NS, EC, C)
    dsts = jnp.broadcast_to(dst.reshape(1, NS, EC, C), (NC, NS, EC, C))
    idx_all = jnp.concatenate([srcs, dsts], axis=-1).reshape(-1)
    zerosHH = jnp.zeros((C, HH), jnp.float32)

    ones2N = jnp.ones((NC * N, HH), jnp.float32)
    deg_wide = _sc_prop(ones2N, idx_all, zerosHH).reshape(NC, NP, HH)[0, :N, :]
    dinv, hp = _tc_first(deg_wide, x, W1)

    hp2 = hp.reshape(NC * N, HH)
    acc = _sc_prop(hp2, idx_all, zerosHH).reshape(NC, NP, HH)[:, :N, :]
    hp = _tc_mid(acc, hp, dinv, b1.reshape(1, H), W2)

    hp2 = hp.reshape(NC * N, HH)
    acc = _sc_prop(hp2, idx_all, zerosHH).reshape(NC, NP, HH)[:, :N, :]
    hp = _tc_mid(acc, hp, dinv, b2.reshape(1, H), W3)

    hp2 = hp.reshape(NC * N, HH)
    acc = _sc_prop(hp2, idx_all, zerosHH).reshape(NC, NP, HH)[:, :N, :]
    return _tc_final(acc, hp, dinv, b3.reshape(1, H),
                     Cw1, Cb1.reshape(1, 512), Cw2, Cb2.reshape(1, 256),
                     Cw3, Cb3.reshape(1, 5))


# speedup vs baseline: 16.5916x; 1.1913x over previous
import functools

import jax
import jax.numpy as jnp
from jax import lax
from jax.experimental import pallas as pl
from jax.experimental.pallas import tpu as pltpu
from jax.experimental.pallas import tpu_sc as plsc

N = 10000
E = 320000
F_IN = 128
H = 256
NC = 2
NS = 16
HH = H // NC
C = 80
EPT = E // NS
EC = EPT // C
EPW = E // (NC * NS)
EDC = EPW // C
NP = 10240
RPT = NP // NS
RW = 128

_MESH = plsc.VectorSubcoreMesh(
    core_axis_name="c", subcore_axis_name="s", num_cores=NC, num_subcores=NS
)


SB = 25
NSB = EC // SB


@functools.partial(
    pl.kernel,
    out_type=jax.ShapeDtypeStruct((NC * NP, HH), jnp.float32),
    mesh=_MESH,
    scratch_types=[
        pltpu.VMEM((SB * 2 * C,), jnp.int32),
        pltpu.VMEM((C, HH), jnp.float32),
        pltpu.VMEM((C, HH), jnp.float32),
        pltpu.VMEM_SHARED((NP, HH), jnp.float32),
        pltpu.SemaphoreType.DMA,
        pltpu.SemaphoreType.DMA,
    ],
)
def _sc_prop(hp2_hbm, idx_hbm, zeros_hbm, out_hbm, idxb, rows0, rows1,
             acc_sh, sem0, sem1):
    c = lax.axis_index("c")
    s = lax.axis_index("s")
    rows = (rows0, rows1)
    sems = (sem0, sem1)
    pltpu.sync_copy(zeros_hbm, rows0)
    for k in range(RPT // C):
        pltpu.sync_copy(rows0, acc_sh.at[pl.ds(s * RPT + k * C, C)])
    plsc.subcore_barrier()
    base = (c * NS + s) * EC

    def outer(o, carry):
        pltpu.sync_copy(
            idx_hbm.at[pl.ds((base + o * SB) * 2 * C, SB * 2 * C)], idxb
        )
        desc = pltpu.async_copy(
            hp2_hbm.at[idxb.at[pl.ds(0, C)]], rows0, sem0
        )
        for j in range(SB):
            if j + 1 < SB:
                nxt = pltpu.async_copy(
                    hp2_hbm.at[idxb.at[pl.ds((j + 1) * 2 * C, C)]],
                    rows[(j + 1) % 2], sems[(j + 1) % 2],
                )
            desc.wait()
            pltpu.sync_copy(
                rows[j % 2], acc_sh.at[idxb.at[pl.ds(j * 2 * C + C, C)]],
                add=True,
            )
            if j + 1 < SB:
                desc = nxt
        return carry

    lax.fori_loop(0, NSB, outer, 0)
    plsc.subcore_barrier()
    for k in range(RPT // C):
        r = s * RPT + k * C
        pltpu.sync_copy(acc_sh.at[pl.ds(r, C)], rows0)
        pltpu.sync_copy(rows0, out_hbm.at[pl.ds(c * NP + r, C)])


SBD = 25
NSBD = EDC // SBD


@functools.partial(
    pl.kernel,
    out_type=jax.ShapeDtypeStruct((NC * NP, HH), jnp.float32),
    mesh=_MESH,
    scratch_types=[
        pltpu.VMEM((SBD * C,), jnp.int32),
        pltpu.VMEM((C, HH), jnp.float32),
        pltpu.VMEM((C, HH), jnp.float32),
        pltpu.VMEM_SHARED((NP, HH), jnp.float32),
    ],
)
def _sc_deg(dst_hbm, ones_hbm, zeros_hbm, out_hbm, idxb, ones_v, zbuf,
            acc_sh):
    c = lax.axis_index("c")
    s = lax.axis_index("s")
    wid = c * NS + s
    pltpu.sync_copy(zeros_hbm, zbuf)
    for k in range(RPT // C):
        pltpu.sync_copy(zbuf, acc_sh.at[pl.ds(s * RPT + k * C, C)])
    pltpu.sync_copy(ones_hbm, ones_v)
    plsc.subcore_barrier()

    def outer(o, carry):
        pltpu.sync_copy(
            dst_hbm.at[pl.ds(wid * EPW + o * SBD * C, SBD * C)], idxb
        )
        for j in range(SBD):
            pltpu.sync_copy(
                ones_v, acc_sh.at[idxb.at[pl.ds(j * C, C)]], add=True
            )
        return carry

    lax.fori_loop(0, NSBD, outer, 0)
    plsc.subcore_barrier()
    for k in range(RPT // C):
        r = s * RPT + k * C
        pltpu.sync_copy(acc_sh.at[pl.ds(r, C)], zbuf)
        pltpu.sync_copy(zbuf, out_hbm.at[pl.ds(c * NP + r, C)])


_BM = 2000
_GRID = N // _BM


def _tc_first_body(deg_ref, x_ref, w_ref, dinv_ref, hp_ref):
    deg = deg_ref[0, :, 0:1] + deg_ref[1, :, 0:1] + 1.0
    dinv = lax.rsqrt(deg)
    dinv_ref[...] = dinv
    xw = jnp.dot(x_ref[...], w_ref[...],
                 preferred_element_type=jnp.float32,
                 precision=lax.Precision.HIGHEST)
    hp_ref[...] = dinv * xw


def _tc_first(deg_wide, x, W1):
    return pl.pallas_call(
        _tc_first_body,
        grid=(_GRID,),
        in_specs=[
            pl.BlockSpec((NC, _BM, HH), lambda i: (0, i, 0)),
            pl.BlockSpec((_BM, F_IN), lambda i: (i, 0)),
            pl.BlockSpec((F_IN, H), lambda i: (0, 0)),
        ],
        out_specs=[
            pl.BlockSpec((_BM, 1), lambda i: (i, 0)),
            pl.BlockSpec((_BM, H), lambda i: (i, 0)),
        ],
        out_shape=[
            jax.ShapeDtypeStruct((N, 1), jnp.float32),
            jax.ShapeDtypeStruct((N, H), jnp.float32),
        ],
    )(deg_wide, x, W1)


def _tc_mid_body(acc_ref, hp_ref, dinv_ref, b_ref, w_ref, out_ref):
    accfull = jnp.concatenate([acc_ref[0], acc_ref[1]], axis=1)
    dinv = dinv_ref[...]
    h = jnp.maximum(dinv * (accfull + hp_ref[...]) + b_ref[...], 0.0)
    hw = jnp.dot(h, w_ref[...], preferred_element_type=jnp.float32,
                 precision=lax.Precision.HIGHEST)
    out_ref[...] = dinv * hw


def _tc_mid(acc, hp, dinv, b_prev, W_next):
    return pl.pallas_call(
        _tc_mid_body,
        grid=(_GRID,),
        in_specs=[
            pl.BlockSpec((NC, _BM, HH), lambda i: (0, i, 0)),
            pl.BlockSpec((_BM, H), lambda i: (i, 0)),
            pl.BlockSpec((_BM, 1), lambda i: (i, 0)),
            pl.BlockSpec((1, H), lambda i: (0, 0)),
            pl.BlockSpec((H, H), lambda i: (0, 0)),
        ],
        out_specs=pl.BlockSpec((_BM, H), lambda i: (i, 0)),
        out_shape=jax.ShapeDtypeStruct((N, H), jnp.float32),
    )(acc, hp, dinv, b_prev, W_next)


def _tc_final_body(acc_ref, hp_ref, dinv_ref, b_ref, cw1_ref, cb1_ref,
                   cw2_ref, cb2_ref, cw3_ref, cb3_ref, out_ref,
                   sum_s, max_s):
    i = pl.program_id(0)
    accfull = jnp.concatenate([acc_ref[0], acc_ref[1]], axis=1)
    h = jnp.maximum(dinv_ref[...] * (accfull + hp_ref[...]) + b_ref[...], 0.0)
    psum = jnp.sum(h, axis=0, keepdims=True)
    pmax = jnp.max(h, axis=0, keepdims=True)

    @pl.when(i == 0)
    def _():
        sum_s[...] = psum
        max_s[...] = pmax

    @pl.when(i > 0)
    def _():
        sum_s[...] += psum
        max_s[...] = jnp.maximum(max_s[...], pmax)

    @pl.when(i == _GRID - 1)
    def _():
        g = jnp.concatenate([sum_s[...] * (1.0 / N), max_s[...]], axis=1)
        z = jnp.maximum(
            jnp.dot(g, cw1_ref[...], preferred_element_type=jnp.float32,
                    precision=lax.Precision.HIGHEST) + cb1_ref[...], 0.0)
        z = jnp.maximum(
            jnp.dot(z, cw2_ref[...], preferred_element_type=jnp.float32,
                    precision=lax.Precision.HIGHEST) + cb2_ref[...], 0.0)
        out_ref[...] = jnp.dot(
            z, cw3_ref[...], preferred_element_type=jnp.float32,
            precision=lax.Precision.HIGHEST) + cb3_ref[...]


def _tc_final(acc, hp, dinv, b3, Cw1, Cb1, Cw2, Cb2, Cw3, Cb3):
    return pl.pallas_call(
        _tc_final_body,
        grid=(_GRID,),
        in_specs=[
            pl.BlockSpec((NC, _BM, HH), lambda i: (0, i, 0)),
            pl.BlockSpec((_BM, H), lambda i: (i, 0)),
            pl.BlockSpec((_BM, 1), lambda i: (i, 0)),
            pl.BlockSpec((1, H), lambda i: (0, 0)),
            pl.BlockSpec((2 * H, 512), lambda i: (0, 0)),
            pl.BlockSpec((1, 512), lambda i: (0, 0)),
            pl.BlockSpec((512, 256), lambda i: (0, 0)),
            pl.BlockSpec((1, 256), lambda i: (0, 0)),
            pl.BlockSpec((256, 5), lambda i: (0, 0)),
            pl.BlockSpec((1, 5), lambda i: (0, 0)),
        ],
        out_specs=pl.BlockSpec((1, 5), lambda i: (0, 0)),
        out_shape=jax.ShapeDtypeStruct((1, 5), jnp.float32),
        scratch_shapes=[
            pltpu.VMEM((1, H), jnp.float32),
            pltpu.VMEM((1, H), jnp.float32),
        ],
    )(acc, hp, dinv, b3, Cw1, Cb1, Cw2, Cb2, Cw3, Cb3)


def kernel(x, edge_index, W1, b1, W2, b2, W3, b3,
           Cw1, Cb1, Cw2, Cb2, Cw3, Cb3):
    src = edge_index[0]
    dst = edge_index[1]
    srcs = jnp.stack([2 * src, 2 * src + 1], axis=0).reshape(NC, NS, EC, C)
    dsts = jnp.broadcast_to(dst.reshape(1, NS, EC, C), (NC, NS, EC, C))
    idx_all = jnp.concatenate([srcs, dsts], axis=-1).reshape(-1)
    zerosHH = jnp.zeros((C, HH), jnp.float32)
    onesHH = jnp.ones((C, HH), jnp.float32)

    deg_wide = _sc_deg(dst, onesHH, zerosHH).reshape(NC, NP, HH)[:, :N, :]
    dinv, hp = _tc_first(deg_wide, x, W1)

    hp2 = hp.reshape(NC * N, HH)
    acc = _sc_prop(hp2, idx_all, zerosHH).reshape(NC, NP, HH)[:, :N, :]
    hp = _tc_mid(acc, hp, dinv, b1.reshape(1, H), W2)

    hp2 = hp.reshape(NC * N, HH)
    acc = _sc_prop(hp2, idx_all, zerosHH).reshape(NC, NP, HH)[:, :N, :]
    hp = _tc_mid(acc, hp, dinv, b2.reshape(1, H), W3)

    hp2 = hp.reshape(NC * N, HH)
    acc = _sc_prop(hp2, idx_all, zerosHH).reshape(NC, NP, HH)[:, :N, :]
    return _tc_final(acc, hp, dinv, b3.reshape(1, H),
                     Cw1, Cb1.reshape(1, 512), Cw2, Cb2.reshape(1, 256),
                     Cw3, Cb3.reshape(1, 5))


# speedup vs baseline: 18.3761x; 1.1076x over previous
import functools

import jax
import jax.numpy as jnp
from jax import lax
from jax.experimental import pallas as pl
from jax.experimental.pallas import tpu as pltpu
from jax.experimental.pallas import tpu_sc as plsc

N = 10000
E = 320000
F_IN = 128
H = 256
NC = 2
NS = 16
HH = H // NC
C = 80
EPT = E // NS
EC = EPT // C
EPW = E // (NC * NS)
EDC = EPW // C
NP = 10240
RPT = NP // NS
RW = 128

_MESH = plsc.VectorSubcoreMesh(
    core_axis_name="c", subcore_axis_name="s", num_cores=NC, num_subcores=NS
)


SB = 25
NSB = EC // SB


@functools.partial(
    pl.kernel,
    out_type=jax.ShapeDtypeStruct((NC * NP, HH), jnp.float32),
    mesh=_MESH,
    scratch_types=[
        pltpu.VMEM((SB * 2 * C,), jnp.int32),
        pltpu.VMEM((C, HH), jnp.float32),
        pltpu.VMEM((C, HH), jnp.float32),
        pltpu.VMEM((C, HH), jnp.float32),
        pltpu.VMEM((C, HH), jnp.float32),
        pltpu.VMEM_SHARED((NP, HH), jnp.float32),
        pltpu.SemaphoreType.DMA,
        pltpu.SemaphoreType.DMA,
        pltpu.SemaphoreType.DMA,
        pltpu.SemaphoreType.DMA,
        pltpu.SemaphoreType.DMA,
        pltpu.SemaphoreType.DMA,
        pltpu.SemaphoreType.DMA,
        pltpu.SemaphoreType.DMA,
    ],
)
def _sc_prop(hp2_hbm, idx_hbm, zeros_hbm, out_hbm, idxb, rows0, rows1,
             rows2, rows3, acc_sh, g0, g1, g2, g3, s0, s1, s2, s3):
    c = lax.axis_index("c")
    s = lax.axis_index("s")
    rows = (rows0, rows1, rows2, rows3)
    gsems = (g0, g1, g2, g3)
    ssems = (s0, s1, s2, s3)
    pltpu.sync_copy(zeros_hbm, rows0)
    for k in range(RPT // C):
        pltpu.sync_copy(rows0, acc_sh.at[pl.ds(s * RPT + k * C, C)])
    plsc.subcore_barrier()
    base = (c * NS + s) * EC

    def _gather(j):
        return pltpu.async_copy(
            hp2_hbm.at[idxb.at[pl.ds(j * 2 * C, C)]],
            rows[j % 4], gsems[j % 4],
        )

    def outer(o, carry):
        pltpu.sync_copy(
            idx_hbm.at[pl.ds((base + o * SB) * 2 * C, SB * 2 * C)], idxb
        )
        gd = {jj: _gather(jj) for jj in range(min(2, SB))}
        sd = {}
        for j in range(SB):
            ja = j + 2
            if ja < SB:
                if ja - 4 >= 0:
                    sd[ja - 4].wait()
                gd[ja] = _gather(ja)
            gd[j].wait()
            sd[j] = pltpu.async_copy(
                rows[j % 4], acc_sh.at[idxb.at[pl.ds(j * 2 * C + C, C)]],
                ssems[j % 4], add=True,
            )
        for j in range(max(0, SB - 4), SB):
            sd[j].wait()
        return carry

    lax.fori_loop(0, NSB, outer, 0)
    plsc.subcore_barrier()
    for k in range(RPT // C):
        r = s * RPT + k * C
        pltpu.sync_copy(acc_sh.at[pl.ds(r, C)], rows0)
        pltpu.sync_copy(rows0, out_hbm.at[pl.ds(c * NP + r, C)])


SBD = 25
NSBD = EDC // SBD


@functools.partial(
    pl.kernel,
    out_type=jax.ShapeDtypeStruct((NC * NP, HH), jnp.float32),
    mesh=_MESH,
    scratch_types=[
        pltpu.VMEM((SBD * C,), jnp.int32),
        pltpu.VMEM((C, HH), jnp.float32),
        pltpu.VMEM((C, HH), jnp.float32),
        pltpu.VMEM_SHARED((NP, HH), jnp.float32),
    ],
)
def _sc_deg(dst_hbm, ones_hbm, zeros_hbm, out_hbm, idxb, ones_v, zbuf,
            acc_sh):
    c = lax.axis_index("c")
    s = lax.axis_index("s")
    wid = c * NS + s
    pltpu.sync_copy(zeros_hbm, zbuf)
    for k in range(RPT // C):
        pltpu.sync_copy(zbuf, acc_sh.at[pl.ds(s * RPT + k * C, C)])
    pltpu.sync_copy(ones_hbm, ones_v)
    plsc.subcore_barrier()

    def outer(o, carry):
        pltpu.sync_copy(
            dst_hbm.at[pl.ds(wid * EPW + o * SBD * C, SBD * C)], idxb
        )
        for j in range(SBD):
            pltpu.sync_copy(
                ones_v, acc_sh.at[idxb.at[pl.ds(j * C, C)]], add=True
            )
        return carry

    lax.fori_loop(0, NSBD, outer, 0)
    plsc.subcore_barrier()
    for k in range(RPT // C):
        r = s * RPT + k * C
        pltpu.sync_copy(acc_sh.at[pl.ds(r, C)], zbuf)
        pltpu.sync_copy(zbuf, out_hbm.at[pl.ds(c * NP + r, C)])


_BM = 2000
_GRID = N // _BM


def _tc_first_body(deg_ref, x_ref, w_ref, dinv_ref, hp_ref):
    deg = deg_ref[0, :, 0:1] + deg_ref[1, :, 0:1] + 1.0
    dinv = lax.rsqrt(deg)
    dinv_ref[...] = dinv
    xw = jnp.dot(x_ref[...], w_ref[...],
                 preferred_element_type=jnp.float32,
                 precision=lax.Precision.HIGHEST)
    hp_ref[...] = dinv * xw


def _tc_first(deg_wide, x, W1):
    return pl.pallas_call(
        _tc_first_body,
        grid=(_GRID,),
        in_specs=[
            pl.BlockSpec((NC, _BM, HH), lambda i: (0, i, 0)),
            pl.BlockSpec((_BM, F_IN), lambda i: (i, 0)),
            pl.BlockSpec((F_IN, H), lambda i: (0, 0)),
        ],
        out_specs=[
            pl.BlockSpec((_BM, 1), lambda i: (i, 0)),
            pl.BlockSpec((_BM, H), lambda i: (i, 0)),
        ],
        out_shape=[
            jax.ShapeDtypeStruct((N, 1), jnp.float32),
            jax.ShapeDtypeStruct((N, H), jnp.float32),
        ],
    )(deg_wide, x, W1)


def _tc_mid_body(acc_ref, hp_ref, dinv_ref, b_ref, w_ref, out_ref):
    accfull = jnp.concatenate([acc_ref[0], acc_ref[1]], axis=1)
    dinv = dinv_ref[...]
    h = jnp.maximum(dinv * (accfull + hp_ref[...]) + b_ref[...], 0.0)
    hw = jnp.dot(h, w_ref[...], preferred_element_type=jnp.float32,
                 precision=lax.Precision.HIGHEST)
    out_ref[...] = dinv * hw


def _tc_mid(acc, hp, dinv, b_prev, W_next):
    return pl.pallas_call(
        _tc_mid_body,
        grid=(_GRID,),
        in_specs=[
            pl.BlockSpec((NC, _BM, HH), lambda i: (0, i, 0)),
            pl.BlockSpec((_BM, H), lambda i: (i, 0)),
            pl.BlockSpec((_BM, 1), lambda i: (i, 0)),
            pl.BlockSpec((1, H), lambda i: (0, 0)),
            pl.BlockSpec((H, H), lambda i: (0, 0)),
        ],
        out_specs=pl.BlockSpec((_BM, H), lambda i: (i, 0)),
        out_shape=jax.ShapeDtypeStruct((N, H), jnp.float32),
    )(acc, hp, dinv, b_prev, W_next)


def _tc_final_body(acc_ref, hp_ref, dinv_ref, b_ref, cw1_ref, cb1_ref,
                   cw2_ref, cb2_ref, cw3_ref, cb3_ref, out_ref,
                   sum_s, max_s):
    i = pl.program_id(0)
    accfull = jnp.concatenate([acc_ref[0], acc_ref[1]], axis=1)
    h = jnp.maximum(dinv_ref[...] * (accfull + hp_ref[...]) + b_ref[...], 0.0)
    psum = jnp.sum(h, axis=0, keepdims=True)
    pmax = jnp.max(h, axis=0, keepdims=True)

    @pl.when(i == 0)
    def _():
        sum_s[...] = psum
        max_s[...] = pmax

    @pl.when(i > 0)
    def _():
        sum_s[...] += psum
        max_s[...] = jnp.maximum(max_s[...], pmax)

    @pl.when(i == _GRID - 1)
    def _():
        g = jnp.concatenate([sum_s[...] * (1.0 / N), max_s[...]], axis=1)
        z = jnp.maximum(
            jnp.dot(g, cw1_ref[...], preferred_element_type=jnp.float32,
                    precision=lax.Precision.HIGHEST) + cb1_ref[...], 0.0)
        z = jnp.maximum(
            jnp.dot(z, cw2_ref[...], preferred_element_type=jnp.float32,
                    precision=lax.Precision.HIGHEST) + cb2_ref[...], 0.0)
        out_ref[...] = jnp.dot(
            z, cw3_ref[...], preferred_element_type=jnp.float32,
            precision=lax.Precision.HIGHEST) + cb3_ref[...]


def _tc_final(acc, hp, dinv, b3, Cw1, Cb1, Cw2, Cb2, Cw3, Cb3):
    return pl.pallas_call(
        _tc_final_body,
        grid=(_GRID,),
        in_specs=[
            pl.BlockSpec((NC, _BM, HH), lambda i: (0, i, 0)),
            pl.BlockSpec((_BM, H), lambda i: (i, 0)),
            pl.BlockSpec((_BM, 1), lambda i: (i, 0)),
            pl.BlockSpec((1, H), lambda i: (0, 0)),
            pl.BlockSpec((2 * H, 512), lambda i: (0, 0)),
            pl.BlockSpec((1, 512), lambda i: (0, 0)),
            pl.BlockSpec((512, 256), lambda i: (0, 0)),
            pl.BlockSpec((1, 256), lambda i: (0, 0)),
            pl.BlockSpec((256, 5), lambda i: (0, 0)),
            pl.BlockSpec((1, 5), lambda i: (0, 0)),
        ],
        out_specs=pl.BlockSpec((1, 5), lambda i: (0, 0)),
        out_shape=jax.ShapeDtypeStruct((1, 5), jnp.float32),
        scratch_shapes=[
            pltpu.VMEM((1, H), jnp.float32),
            pltpu.VMEM((1, H), jnp.float32),
        ],
    )(acc, hp, dinv, b3, Cw1, Cb1, Cw2, Cb2, Cw3, Cb3)


def kernel(x, edge_index, W1, b1, W2, b2, W3, b3,
           Cw1, Cb1, Cw2, Cb2, Cw3, Cb3):
    src = edge_index[0]
    dst = edge_index[1]
    srcs = jnp.stack([2 * src, 2 * src + 1], axis=0).reshape(NC, NS, EC, C)
    dsts = jnp.broadcast_to(dst.reshape(1, NS, EC, C), (NC, NS, EC, C))
    idx_all = jnp.concatenate([srcs, dsts], axis=-1).reshape(-1)
    zerosHH = jnp.zeros((C, HH), jnp.float32)
    onesHH = jnp.ones((C, HH), jnp.float32)

    deg_wide = _sc_deg(dst, onesHH, zerosHH).reshape(NC, NP, HH)[:, :N, :]
    dinv, hp = _tc_first(deg_wide, x, W1)

    hp2 = hp.reshape(NC * N, HH)
    acc = _sc_prop(hp2, idx_all, zerosHH).reshape(NC, NP, HH)[:, :N, :]
    hp = _tc_mid(acc, hp, dinv, b1.reshape(1, H), W2)

    hp2 = hp.reshape(NC * N, HH)
    acc = _sc_prop(hp2, idx_all, zerosHH).reshape(NC, NP, HH)[:, :N, :]
    hp = _tc_mid(acc, hp, dinv, b2.reshape(1, H), W3)

    hp2 = hp.reshape(NC * N, HH)
    acc = _sc_prop(hp2, idx_all, zerosHH).reshape(NC, NP, HH)[:, :N, :]
    return _tc_final(acc, hp, dinv, b3.reshape(1, H),
                     Cw1, Cb1.reshape(1, 512), Cw2, Cb2.reshape(1, 256),
                     Cw3, Cb3.reshape(1, 5))
